# baseline (speedup 1.0000x reference)
import functools

import jax
import jax.numpy as jnp
from jax import lax
from jax.experimental import pallas as pl
from jax.experimental.pallas import tpu as pltpu
from jax.experimental.pallas import tpu_sc as plsc

N = 10000
NP = 10240
E = 160000
D = 128
DH = D // 2
NC = 2
NS = 16
G = 80
B = 125
NBUF = 6
RPT = NP // NS
ZB = 128
ZC = RPT // ZB


def _sc_pipeline(groups, m_hbm, src_v, dst_v, rows_v, acc, sem_g, sem_s):
    for j in range(NBUF - 1):
        pltpu.async_copy(m_hbm.at[src_v.at[j]], rows_v.at[j], sem_g)

    def body(h, carry):
        for j in range(NBUF):
            g = NBUF * h + j
            buf = rows_v.at[j]
            pltpu.make_async_copy(m_hbm.at[src_v.at[g]], buf, sem_g).wait()
            pltpu.async_copy(buf, acc.at[dst_v.at[g]], sem_s, add=True)

            @pl.when(g >= 1)
            def _():
                pltpu.make_async_copy(
                    rows_v.at[(j + NBUF - 1) % NBUF],
                    acc.at[dst_v.at[g - 1]], sem_s).wait()

            @pl.when(g + NBUF - 1 < groups)
            def _():
                pltpu.async_copy(m_hbm.at[src_v.at[g + NBUF - 1]],
                                 rows_v.at[(j + NBUF - 1) % NBUF], sem_g)
        return carry

    lax.fori_loop(0, groups // NBUF, body, 0)
    for g in range(groups - groups % NBUF, groups):
        j = g % NBUF
        buf = rows_v.at[j]
        pltpu.make_async_copy(m_hbm.at[src_v.at[g]], buf, sem_g).wait()
        pltpu.async_copy(buf, acc.at[dst_v.at[g]], sem_s, add=True)
        pltpu.make_async_copy(rows_v.at[(j + NBUF - 1) % NBUF],
                              acc.at[dst_v.at[g - 1]], sem_s).wait()
        if g + NBUF - 1 < groups:
            pltpu.async_copy(m_hbm.at[src_v.at[g + NBUF - 1]],
                             rows_v.at[(j + NBUF - 1) % NBUF], sem_g)
    pltpu.make_async_copy(rows_v.at[(groups - 1) % NBUF],
                          acc.at[dst_v.at[groups - 1]], sem_s).wait()


@functools.lru_cache(maxsize=None)
def _make_sc_two_dir():
    mesh = plsc.VectorSubcoreMesh(core_axis_name="c", subcore_axis_name="s")

    @functools.partial(
        pl.kernel,
        mesh=mesh,
        compiler_params=pltpu.CompilerParams(use_tc_tiling_on_sc=False),
        out_type=(jax.ShapeDtypeStruct((2, NP, DH), jnp.float32),
                  jax.ShapeDtypeStruct((2, NP, DH), jnp.float32)),
        scratch_types=[
            pltpu.VMEM((G, B), jnp.int32),
            pltpu.VMEM((G, B), jnp.int32),
            pltpu.VMEM((NBUF, B, DH), jnp.float32),
            pltpu.VMEM((ZB, DH), jnp.float32),
            pltpu.VMEM((ZB, DH), jnp.float32),
            pltpu.VMEM_SHARED((NP, DH), jnp.float32),
            pltpu.SemaphoreType.DMA,
            pltpu.SemaphoreType.DMA,
        ],
    )
    def k(mlo_hbm, mhi_hbm, src0_hbm, dst0_hbm, src1_hbm, dst1_hbm, z_hbm,
          olo_hbm, ohi_hbm, src_v, dst_v, rows_v, zw_v, wout_v, acc,
          sem_g, sem_s):
        cid = lax.axis_index("c")
        sid = lax.axis_index("s")

        @pl.when(cid == 0)
        def _():
            pltpu.sync_copy(src0_hbm.at[sid], src_v)
            pltpu.sync_copy(dst0_hbm.at[sid], dst_v)

        @pl.when(cid == 1)
        def _():
            pltpu.sync_copy(src1_hbm.at[sid], src_v)
            pltpu.sync_copy(dst1_hbm.at[sid], dst_v)

        pltpu.sync_copy(z_hbm, zw_v)

        def one_pass(m_hbm, out_hbm):
            for c in range(ZC):
                pltpu.sync_copy(zw_v, acc.at[pl.ds(sid * RPT + c * ZB, ZB)])
            plsc.subcore_barrier()
            _sc_pipeline(G, m_hbm, src_v, dst_v, rows_v, acc, sem_g, sem_s)
            plsc.subcore_barrier()
            for c in range(ZC):
                pltpu.sync_copy(acc.at[pl.ds(sid * RPT + c * ZB, ZB)], wout_v)
                pltpu.sync_copy(
                    wout_v, out_hbm.at[1 - cid, pl.ds(sid * RPT + c * ZB, ZB)])

        one_pass(mlo_hbm, olo_hbm)
        one_pass(mhi_hbm, ohi_hbm)

    return k


@functools.lru_cache(maxsize=None)
def _make_sc_one_dir():
    mesh = plsc.VectorSubcoreMesh(core_axis_name="c", subcore_axis_name="s")
    G2 = G // 2

    @functools.partial(
        pl.kernel,
        mesh=mesh,
        compiler_params=pltpu.CompilerParams(use_tc_tiling_on_sc=False),
        out_type=(jax.ShapeDtypeStruct((2, NP, DH), jnp.float32),
                  jax.ShapeDtypeStruct((2, NP, DH), jnp.float32)),
        scratch_types=[
            pltpu.VMEM((G2, B), jnp.int32),
            pltpu.VMEM((G2, B), jnp.int32),
            pltpu.VMEM((NBUF, B, DH), jnp.float32),
            pltpu.VMEM((ZB, DH), jnp.float32),
            pltpu.VMEM((ZB, DH), jnp.float32),
            pltpu.VMEM_SHARED((NP, DH), jnp.float32),
            pltpu.SemaphoreType.DMA,
            pltpu.SemaphoreType.DMA,
        ],
    )
    def k(mlo_hbm, mhi_hbm, src_hbm, dst_hbm, z_hbm, olo_hbm, ohi_hbm,
          src_v, dst_v, rows_v, zw_v, wout_v, acc, sem_g, sem_s):
        cid = lax.axis_index("c")
        sid = lax.axis_index("s")
        pltpu.sync_copy(src_hbm.at[cid, sid], src_v)
        pltpu.sync_copy(dst_hbm.at[cid, sid], dst_v)
        pltpu.sync_copy(z_hbm, zw_v)

        def one_pass(m_hbm, out_hbm):
            for c in range(ZC):
                pltpu.sync_copy(zw_v, acc.at[pl.ds(sid * RPT + c * ZB, ZB)])
            plsc.subcore_barrier()
            _sc_pipeline(G2, m_hbm, src_v, dst_v, rows_v, acc, sem_g, sem_s)
            plsc.subcore_barrier()
            for c in range(ZC):
                pltpu.sync_copy(acc.at[pl.ds(sid * RPT + c * ZB, ZB)], wout_v)
                pltpu.sync_copy(
                    wout_v, out_hbm.at[cid, pl.ds(sid * RPT + c * ZB, ZB)])

        one_pass(mlo_hbm, olo_hbm)
        one_pass(mhi_hbm, ohi_hbm)

    return k


def _pair_weights(Wm1, bm1, Wm2, bm2, Wu1, bu1, Wu2, bu2, Wo, bo):
    z = jnp.zeros
    f32 = jnp.float32

    def blkdiag(Wtop, Wbot, r, c):
        out = z((2 * r, 2 * c), f32)
        out = out.at[:r, :c].set(Wtop)
        out = out.at[r:, c:].set(Wbot)
        return out

    p = {}
    p["M1x"] = blkdiag(Wm1, Wm1, D, 32)
    p["U1a"] = blkdiag(Wu1[:DH], Wu1[:DH], DH, 16)
    p["U1b"] = blkdiag(Wu1[DH:], Wu1[DH:], DH, 16)
    p["b1"] = jnp.concatenate([bu1, bu1]).reshape(1, -1)
    p["U2a"] = blkdiag(Wu2[:, :DH], Wu2[:, :DH], 16, DH)
    p["U2b"] = blkdiag(Wu2[:, DH:], Wu2[:, DH:], 16, DH)
    p["b2a"] = jnp.concatenate([bu2[:DH], bu2[:DH]]).reshape(1, -1)
    p["b2b"] = jnp.concatenate([bu2[DH:], bu2[DH:]]).reshape(1, -1)
    p["M1a"] = blkdiag(Wm1[:DH], Wm1[:DH], DH, 32)
    p["M1b"] = blkdiag(Wm1[DH:], Wm1[DH:], DH, 32)
    p["b3"] = jnp.concatenate([bm1, bm1]).reshape(1, -1)
    p["M2a"] = blkdiag(Wm2[:, :DH], Wm2[:, :DH], 32, DH)
    p["M2b"] = blkdiag(Wm2[:, DH:], Wm2[:, DH:], 32, DH)
    p["b4a"] = jnp.concatenate([bm2[:DH], bm2[:DH]]).reshape(1, -1)
    p["b4b"] = jnp.concatenate([bm2[DH:], bm2[DH:]]).reshape(1, -1)
    p["Oa"] = blkdiag(Wo[:DH], Wo[:DH], DH, D)
    p["Ob"] = blkdiag(Wo[DH:], Wo[DH:], DH, D)
    p["bo"] = jnp.concatenate([bo, bo]).reshape(1, -1)
    return p


def _dot(a, b):
    return jnp.dot(a, b, preferred_element_type=jnp.float32)


_GRID = 8


def _wspec(shape):
    return pl.BlockSpec(shape, lambda i: (0, 0))


def _msg_tc(x2, p):
    def body(x_ref, m1x, b3, m2a, m2b, b4a, b4b, ol, oh):
        hm = jnp.maximum(_dot(x_ref[...], m1x[...]) + b3[...], 0.0)
        ol[...] = jnp.maximum(_dot(hm, m2a[...]) + b4a[...], 0.0)
        oh[...] = jnp.maximum(_dot(hm, m2b[...]) + b4b[...], 0.0)
    r = x2.shape[0]
    br = r // _GRID
    rspec = pl.BlockSpec((br, D), lambda i: (i, 0))
    return pl.pallas_call(
        body,
        grid=(_GRID,),
        in_specs=[pl.BlockSpec((br, 2 * D), lambda i: (i, 0)),
                  _wspec((2 * D, 64)), _wspec((1, 64)),
                  _wspec((64, D)), _wspec((64, D)),
                  _wspec((1, D)), _wspec((1, D))],
        out_specs=(rspec, rspec),
        out_shape=(jax.ShapeDtypeStruct((r, D), jnp.float32),
                   jax.ShapeDtypeStruct((r, D), jnp.float32)),
    )(x2, p["M1x"], p["b3"], p["M2a"], p["M2b"], p["b4a"], p["b4b"])


def _updmsg_tc(a_lo, a_hi, p):
    def body(al, ah, u1a, u1b, b1, u2a, u2b, b2a, b2b,
             m1a, m1b, b3, m2a, m2b, b4a, b4b, ol, oh):
        h = jnp.maximum(_dot(al[...], u1a[...]) + _dot(ah[...], u1b[...])
                        + b1[...], 0.0)
        xl = jnp.maximum(_dot(h, u2a[...]) + b2a[...], 0.0)
        xh = jnp.maximum(_dot(h, u2b[...]) + b2b[...], 0.0)
        hm = jnp.maximum(_dot(xl, m1a[...]) + _dot(xh, m1b[...])
                         + b3[...], 0.0)
        ol[...] = jnp.maximum(_dot(hm, m2a[...]) + b4a[...], 0.0)
        oh[...] = jnp.maximum(_dot(hm, m2b[...]) + b4b[...], 0.0)
    r = a_lo.shape[0]
    br = r // _GRID
    rspec = pl.BlockSpec((br, D), lambda i: (i, 0))
    return pl.pallas_call(
        body,
        grid=(_GRID,),
        in_specs=[rspec, rspec,
                  _wspec((D, 32)), _wspec((D, 32)), _wspec((1, 32)),
                  _wspec((32, D)), _wspec((32, D)),
                  _wspec((1, D)), _wspec((1, D)),
                  _wspec((D, 64)), _wspec((D, 64)), _wspec((1, 64)),
                  _wspec((64, D)), _wspec((64, D)),
                  _wspec((1, D)), _wspec((1, D))],
        out_specs=(rspec, rspec),
        out_shape=(jax.ShapeDtypeStruct((r, D), jnp.float32),
                   jax.ShapeDtypeStruct((r, D), jnp.float32)),
    )(a_lo, a_hi, p["U1a"], p["U1b"], p["b1"], p["U2a"], p["U2b"],
      p["b2a"], p["b2b"], p["M1a"], p["M1b"], p["b3"], p["M2a"], p["M2b"],
      p["b4a"], p["b4b"])


def _head_tc(pp_lo, pp_hi, p):
    def body(al0, al1, ah0, ah1, u1a, u1b, b1, u2a, u2b, b2a, b2b,
             oa, ob, bo_, o_ref):
        al = al0[...] + al1[...]
        ah = ah0[...] + ah1[...]
        h = jnp.maximum(_dot(al, u1a[...]) + _dot(ah, u1b[...])
                        + b1[...], 0.0)
        xl = jnp.maximum(_dot(h, u2a[...]) + b2a[...], 0.0)
        xh = jnp.maximum(_dot(h, u2b[...]) + b2b[...], 0.0)
        o_ref[...] = jnp.tanh(_dot(xl, oa[...]) + _dot(xh, ob[...])
                              + bo_[...])
    br = NP // 2 // _GRID
    spec0 = pl.BlockSpec((br, D), lambda i: (i, 0))
    spec1 = pl.BlockSpec((br, D), lambda i: (i + _GRID, 0))
    return pl.pallas_call(
        body,
        grid=(_GRID,),
        in_specs=[spec0, spec1, spec0, spec1,
                  _wspec((D, 32)), _wspec((D, 32)), _wspec((1, 32)),
                  _wspec((32, D)), _wspec((32, D)),
                  _wspec((1, D)), _wspec((1, D)),
                  _wspec((D, 2 * D)), _wspec((D, 2 * D)), _wspec((1, 2 * D))],
        out_specs=pl.BlockSpec((br, 2 * D), lambda i: (i, 0)),
        out_shape=jax.ShapeDtypeStruct((NP // 2, 2 * D), jnp.float32),
    )(pp_lo, pp_lo, pp_hi, pp_hi, p["U1a"], p["U1b"], p["b1"],
      p["U2a"], p["U2b"], p["b2a"], p["b2b"], p["Oa"], p["Ob"], p["bo"])


def kernel(x_served, x_interfered, edge_index_s2i, edge_index_i2s,
           Wm1, bm1, Wm2, bm2, Wu1, bu1, Wu2, bu2, Wo, bo):
    e_s2i = edge_index_s2i.astype(jnp.int32)
    e_i2s = edge_index_i2s.astype(jnp.int32)
    src0 = e_s2i[0].reshape(NS, G, B)
    dst0 = e_s2i[1].reshape(NS, G, B)
    src1 = (e_i2s[0] + NP).reshape(NS, G, B)
    dst1 = e_i2s[1].reshape(NS, G, B)
    zeros = jnp.zeros((ZB, DH), jnp.float32)
    pad = jnp.zeros((NP - N, D), jnp.float32)
    p = _pair_weights(Wm1, bm1, Wm2, bm2, Wu1, bu1, Wu2, bu2, Wo, bo)

    pad2 = pad.reshape((NP - N) // 2, 2 * D)
    x2 = jnp.concatenate([x_served.reshape(N // 2, 2 * D), pad2,
                          x_interfered.reshape(N // 2, 2 * D), pad2])

    m_lo, m_hi = _msg_tc(x2, p)
    sc2 = _make_sc_two_dir()
    for _ in range(2):
        a_lo, a_hi = sc2(m_lo.reshape(2 * NP, DH), m_hi.reshape(2 * NP, DH),
                         src0, dst0, src1, dst1, zeros)
        m_lo, m_hi = _updmsg_tc(a_lo.reshape(NP, D), a_hi.reshape(NP, D), p)
    p_lo, p_hi = _make_sc_one_dir()(
        m_lo.reshape(2 * NP, DH), m_hi.reshape(2 * NP, DH),
        src1.reshape(2, NS, G // 2, B), dst1.reshape(2, NS, G // 2, B), zeros)
    out_pair = _head_tc(p_lo.reshape(NP, D), p_hi.reshape(NP, D), p)
    return out_pair.reshape(NP, D)[:N]


# speedup vs baseline: 1.0411x; 1.0411x over previous
import functools

import jax
import jax.numpy as jnp
from jax import lax
from jax.experimental import pallas as pl
from jax.experimental.pallas import tpu as pltpu
from jax.experimental.pallas import tpu_sc as plsc

N = 10000
NP = 10240
E = 160000
D = 128
DH = D // 2
NC = 2
NS = 16
G = 80
B = 125
NBUF = 6
RPT = NP // NS
ZB = 128
ZC = RPT // ZB


def _sc_pipeline(groups, m_hbm, src_v, dst_v, rows_v, acc, sem_g, sem_s):
    for j in range(NBUF - 1):
        pltpu.async_copy(m_hbm.at[src_v.at[j]], rows_v.at[j], sem_g)

    def body(h, carry):
        for j in range(NBUF):
            g = NBUF * h + j
            buf = rows_v.at[j]
            pltpu.make_async_copy(m_hbm.at[src_v.at[g]], buf, sem_g).wait()
            pltpu.async_copy(buf, acc.at[dst_v.at[g]], sem_s, add=True)

            @pl.when(g >= 1)
            def _():
                pltpu.make_async_copy(
                    rows_v.at[(j + NBUF - 1) % NBUF],
                    acc.at[dst_v.at[g - 1]], sem_s).wait()

            @pl.when(g + NBUF - 1 < groups)
            def _():
                pltpu.async_copy(m_hbm.at[src_v.at[g + NBUF - 1]],
                                 rows_v.at[(j + NBUF - 1) % NBUF], sem_g)
        return carry

    lax.fori_loop(0, groups // NBUF, body, 0)
    for g in range(groups - groups % NBUF, groups):
        j = g % NBUF
        buf = rows_v.at[j]
        pltpu.make_async_copy(m_hbm.at[src_v.at[g]], buf, sem_g).wait()
        pltpu.async_copy(buf, acc.at[dst_v.at[g]], sem_s, add=True)
        pltpu.make_async_copy(rows_v.at[(j + NBUF - 1) % NBUF],
                              acc.at[dst_v.at[g - 1]], sem_s).wait()
        if g + NBUF - 1 < groups:
            pltpu.async_copy(m_hbm.at[src_v.at[g + NBUF - 1]],
                             rows_v.at[(j + NBUF - 1) % NBUF], sem_g)
    pltpu.make_async_copy(rows_v.at[(groups - 1) % NBUF],
                          acc.at[dst_v.at[groups - 1]], sem_s).wait()


@functools.lru_cache(maxsize=None)
def _make_sc_two_dir():
    mesh = plsc.VectorSubcoreMesh(core_axis_name="c", subcore_axis_name="s")

    @functools.partial(
        pl.kernel,
        mesh=mesh,
        compiler_params=pltpu.CompilerParams(use_tc_tiling_on_sc=False),
        out_type=(jax.ShapeDtypeStruct((2, NP, DH), jnp.float32),
                  jax.ShapeDtypeStruct((2, NP, DH), jnp.float32)),
        scratch_types=[
            pltpu.VMEM((G, B), jnp.int32),
            pltpu.VMEM((G, B), jnp.int32),
            pltpu.VMEM((NBUF, B, DH), jnp.float32),
            pltpu.VMEM((ZB, DH), jnp.float32),
            pltpu.VMEM((ZB, DH), jnp.float32),
            pltpu.VMEM_SHARED((NP, DH), jnp.float32),
            pltpu.SemaphoreType.DMA,
            pltpu.SemaphoreType.DMA,
        ],
    )
    def k(mlo_hbm, mhi_hbm, src0_hbm, dst0_hbm, src1_hbm, dst1_hbm, z_hbm,
          olo_hbm, ohi_hbm, src_v, dst_v, rows_v, zw_v, wout_v, acc,
          sem_g, sem_s):
        cid = lax.axis_index("c")
        sid = lax.axis_index("s")

        @pl.when(cid == 0)
        def _():
            pltpu.sync_copy(src0_hbm.at[sid], src_v)
            pltpu.sync_copy(dst0_hbm.at[sid], dst_v)

        @pl.when(cid == 1)
        def _():
            pltpu.sync_copy(src1_hbm.at[sid], src_v)
            pltpu.sync_copy(dst1_hbm.at[sid], dst_v)

        pltpu.sync_copy(z_hbm, zw_v)

        def one_pass(m_hbm, out_hbm):
            for c in range(ZC):
                pltpu.sync_copy(zw_v, acc.at[pl.ds(sid * RPT + c * ZB, ZB)])
            plsc.subcore_barrier()
            _sc_pipeline(G, m_hbm, src_v, dst_v, rows_v, acc, sem_g, sem_s)
            plsc.subcore_barrier()
            for c in range(ZC):
                pltpu.sync_copy(acc.at[pl.ds(sid * RPT + c * ZB, ZB)], wout_v)
                pltpu.sync_copy(
                    wout_v, out_hbm.at[1 - cid, pl.ds(sid * RPT + c * ZB, ZB)])

        one_pass(mlo_hbm, olo_hbm)
        one_pass(mhi_hbm, ohi_hbm)

    return k


@functools.lru_cache(maxsize=None)
def _make_sc_one_dir():
    mesh = plsc.VectorSubcoreMesh(core_axis_name="c", subcore_axis_name="s")
    G2 = G // 2

    @functools.partial(
        pl.kernel,
        mesh=mesh,
        compiler_params=pltpu.CompilerParams(use_tc_tiling_on_sc=False),
        out_type=(jax.ShapeDtypeStruct((2, NP, DH), jnp.float32),
                  jax.ShapeDtypeStruct((2, NP, DH), jnp.float32)),
        scratch_types=[
            pltpu.VMEM((G2, B), jnp.int32),
            pltpu.VMEM((G2, B), jnp.int32),
            pltpu.VMEM((NBUF, B, DH), jnp.float32),
            pltpu.VMEM((ZB, DH), jnp.float32),
            pltpu.VMEM((ZB, DH), jnp.float32),
            pltpu.VMEM_SHARED((NP, DH), jnp.float32),
            pltpu.SemaphoreType.DMA,
            pltpu.SemaphoreType.DMA,
        ],
    )
    def k(mlo_hbm, mhi_hbm, src_hbm, dst_hbm, z_hbm, olo_hbm, ohi_hbm,
          src_v, dst_v, rows_v, zw_v, wout_v, acc, sem_g, sem_s):
        cid = lax.axis_index("c")
        sid = lax.axis_index("s")
        pltpu.sync_copy(src_hbm.at[cid, sid], src_v)
        pltpu.sync_copy(dst_hbm.at[cid, sid], dst_v)
        pltpu.sync_copy(z_hbm, zw_v)

        def one_pass(m_hbm, out_hbm):
            for c in range(ZC):
                pltpu.sync_copy(zw_v, acc.at[pl.ds(sid * RPT + c * ZB, ZB)])
            plsc.subcore_barrier()
            _sc_pipeline(G2, m_hbm, src_v, dst_v, rows_v, acc, sem_g, sem_s)
            plsc.subcore_barrier()
            for c in range(ZC):
                pltpu.sync_copy(acc.at[pl.ds(sid * RPT + c * ZB, ZB)], wout_v)
                pltpu.sync_copy(
                    wout_v, out_hbm.at[cid, pl.ds(sid * RPT + c * ZB, ZB)])

        one_pass(mlo_hbm, olo_hbm)
        one_pass(mhi_hbm, ohi_hbm)

    return k


def _pair_weights(Wm1, bm1, Wm2, bm2, Wu1, bu1, Wu2, bu2, Wo, bo):
    z = jnp.zeros
    f32 = jnp.float32

    def blkdiag(Wtop, Wbot, r, c):
        out = z((2 * r, 2 * c), f32)
        out = out.at[:r, :c].set(Wtop)
        out = out.at[r:, c:].set(Wbot)
        return out

    p = {}
    p["M1x"] = blkdiag(Wm1, Wm1, D, 32)
    p["U1a"] = blkdiag(Wu1[:DH], Wu1[:DH], DH, 16)
    p["U1b"] = blkdiag(Wu1[DH:], Wu1[DH:], DH, 16)
    p["b1"] = jnp.concatenate([bu1, bu1]).reshape(1, -1)
    p["U2a"] = blkdiag(Wu2[:, :DH], Wu2[:, :DH], 16, DH)
    p["U2b"] = blkdiag(Wu2[:, DH:], Wu2[:, DH:], 16, DH)
    p["b2a"] = jnp.concatenate([bu2[:DH], bu2[:DH]]).reshape(1, -1)
    p["b2b"] = jnp.concatenate([bu2[DH:], bu2[DH:]]).reshape(1, -1)
    p["M1a"] = blkdiag(Wm1[:DH], Wm1[:DH], DH, 32)
    p["M1b"] = blkdiag(Wm1[DH:], Wm1[DH:], DH, 32)
    p["b3"] = jnp.concatenate([bm1, bm1]).reshape(1, -1)
    p["M2a"] = blkdiag(Wm2[:, :DH], Wm2[:, :DH], 32, DH)
    p["M2b"] = blkdiag(Wm2[:, DH:], Wm2[:, DH:], 32, DH)
    p["b4a"] = jnp.concatenate([bm2[:DH], bm2[:DH]]).reshape(1, -1)
    p["b4b"] = jnp.concatenate([bm2[DH:], bm2[DH:]]).reshape(1, -1)
    p["Oa"] = blkdiag(Wo[:DH], Wo[:DH], DH, D)
    p["Ob"] = blkdiag(Wo[DH:], Wo[DH:], DH, D)
    p["bo"] = jnp.concatenate([bo, bo]).reshape(1, -1)
    return p


def _dot(a, b):
    return jnp.dot(a, b, preferred_element_type=jnp.float32)


def _msg_tc(x2, p):
    def body(x_ref, m1x, b3, m2a, m2b, b4a, b4b, ol, oh):
        hm = jnp.maximum(_dot(x_ref[...], m1x[...]) + b3[...], 0.0)
        ol[...] = jnp.maximum(_dot(hm, m2a[...]) + b4a[...], 0.0)
        oh[...] = jnp.maximum(_dot(hm, m2b[...]) + b4b[...], 0.0)
    r = x2.shape[0]
    return pl.pallas_call(
        body,
        out_shape=(jax.ShapeDtypeStruct((r, D), jnp.float32),
                   jax.ShapeDtypeStruct((r, D), jnp.float32)),
    )(x2, p["M1x"], p["b3"], p["M2a"], p["M2b"], p["b4a"], p["b4b"])


def _updmsg_tc(a_lo, a_hi, p):
    def body(al, ah, u1a, u1b, b1, u2a, u2b, b2a, b2b,
             m1a, m1b, b3, m2a, m2b, b4a, b4b, ol, oh):
        h = jnp.maximum(_dot(al[...], u1a[...]) + _dot(ah[...], u1b[...])
                        + b1[...], 0.0)
        xl = jnp.maximum(_dot(h, u2a[...]) + b2a[...], 0.0)
        xh = jnp.maximum(_dot(h, u2b[...]) + b2b[...], 0.0)
        hm = jnp.maximum(_dot(xl, m1a[...]) + _dot(xh, m1b[...])
                         + b3[...], 0.0)
        ol[...] = jnp.maximum(_dot(hm, m2a[...]) + b4a[...], 0.0)
        oh[...] = jnp.maximum(_dot(hm, m2b[...]) + b4b[...], 0.0)
    r = a_lo.shape[0]
    return pl.pallas_call(
        body,
        out_shape=(jax.ShapeDtypeStruct((r, D), jnp.float32),
                   jax.ShapeDtypeStruct((r, D), jnp.float32)),
    )(a_lo, a_hi, p["U1a"], p["U1b"], p["b1"], p["U2a"], p["U2b"],
      p["b2a"], p["b2b"], p["M1a"], p["M1b"], p["b3"], p["M2a"], p["M2b"],
      p["b4a"], p["b4b"])


def _head_tc(pp_lo, pp_hi, p):
    def body(al_ref, ah_ref, u1a, u1b, b1, u2a, u2b, b2a, b2b,
             oa, ob, bo_, o_ref):
        al = al_ref[: NP // 2, :] + al_ref[NP // 2:, :]
        ah = ah_ref[: NP // 2, :] + ah_ref[NP // 2:, :]
        h = jnp.maximum(_dot(al, u1a[...]) + _dot(ah, u1b[...])
                        + b1[...], 0.0)
        xl = jnp.maximum(_dot(h, u2a[...]) + b2a[...], 0.0)
        xh = jnp.maximum(_dot(h, u2b[...]) + b2b[...], 0.0)
        o_ref[...] = jnp.tanh(_dot(xl, oa[...]) + _dot(xh, ob[...])
                              + bo_[...])
    return pl.pallas_call(
        body,
        out_shape=jax.ShapeDtypeStruct((NP // 2, 2 * D), jnp.float32),
    )(pp_lo, pp_hi, p["U1a"], p["U1b"], p["b1"], p["U2a"], p["U2b"],
      p["b2a"], p["b2b"], p["Oa"], p["Ob"], p["bo"])


def kernel(x_served, x_interfered, edge_index_s2i, edge_index_i2s,
           Wm1, bm1, Wm2, bm2, Wu1, bu1, Wu2, bu2, Wo, bo):
    e_s2i = edge_index_s2i.astype(jnp.int32)
    e_i2s = edge_index_i2s.astype(jnp.int32)
    src0 = e_s2i[0].reshape(NS, G, B)
    dst0 = e_s2i[1].reshape(NS, G, B)
    src1 = (e_i2s[0] + NP).reshape(NS, G, B)
    dst1 = e_i2s[1].reshape(NS, G, B)
    zeros = jnp.zeros((ZB, DH), jnp.float32)
    pad = jnp.zeros((NP - N, D), jnp.float32)
    p = _pair_weights(Wm1, bm1, Wm2, bm2, Wu1, bu1, Wu2, bu2, Wo, bo)

    x2 = jnp.concatenate([x_served, pad, x_interfered, pad]).reshape(NP, 2 * D)

    m_lo, m_hi = _msg_tc(x2, p)
    sc2 = _make_sc_two_dir()
    for _ in range(2):
        a_lo, a_hi = sc2(m_lo.reshape(2 * NP, DH), m_hi.reshape(2 * NP, DH),
                         src0, dst0, src1, dst1, zeros)
        m_lo, m_hi = _updmsg_tc(a_lo.reshape(NP, D), a_hi.reshape(NP, D), p)
    p_lo, p_hi = _make_sc_one_dir()(
        m_lo.reshape(2 * NP, DH), m_hi.reshape(2 * NP, DH),
        src1.reshape(2, NS, G // 2, B), dst1.reshape(2, NS, G // 2, B), zeros)
    out_pair = _head_tc(p_lo.reshape(NP, D), p_hi.reshape(NP, D), p)
    return out_pair.reshape(NP, D)[:N]
